# Initial kernel scaffold; baseline (speedup 1.0000x reference)
import jax
import jax.numpy as jnp
from jax.experimental import pallas as pl


def kernel(x, edge_index, W1_0, b1_0, W2_0, b2_0, W1_1, b1_1, W2_1, b2_1, W1_2, b1_2, W2_2, b2_2, Wp, bp):
    raise NotImplementedError("write your pallas kernel here")


# speedup vs baseline: 3.4517x; 3.4517x over previous
import functools

import jax
import jax.numpy as jnp
from jax import lax
from jax.experimental import pallas as pl
from jax.experimental.pallas import tpu as pltpu
from jax.experimental.pallas import tpu_sc as plsc

N = 10000
E = 320000
H = 128

NC = 2
NS = 16
EC = E // NC
ET = EC // NS
CH = 80
NCHUNK = ET // CH
ZR = 624
ZTAIL = N - NS * ZR

_mesh = plsc.VectorSubcoreMesh(
    core_axis_name="c", subcore_axis_name="s", num_cores=NC, num_subcores=NS
)


@functools.partial(
    pl.kernel,
    out_type=jax.ShapeDtypeStruct((NC, N, H), jnp.float32),
    mesh=_mesh,
    scratch_types=[
        pltpu.VMEM((CH,), jnp.int32),
        pltpu.VMEM((CH,), jnp.int32),
        pltpu.VMEM((CH, H), jnp.float32),
        pltpu.VMEM_SHARED((N, H), jnp.float32),
        pltpu.SemaphoreType.DMA,
    ],
)
def _segsum(tab, src, dst, zeros, out, src_v, dst_v, rows_v, acc, sem):
    cid = lax.axis_index("c")
    sid = lax.axis_index("s")
    r0 = pl.multiple_of(sid * ZR, 8)
    pltpu.sync_copy(zeros.at[pl.ds(r0, ZR)], acc.at[pl.ds(r0, ZR)])

    @pl.when(sid == 0)
    def _():
        pltpu.sync_copy(zeros.at[pl.ds(NS * ZR, ZTAIL)],
                        acc.at[pl.ds(NS * ZR, ZTAIL)])

    plsc.subcore_barrier()
    base0 = cid * EC + sid * ET

    def chunk(j, carry):
        base = pl.multiple_of(base0 + j * CH, 8)
        pltpu.sync_copy(src.at[pl.ds(base, CH)], src_v)
        pltpu.sync_copy(dst.at[pl.ds(base, CH)], dst_v)
        pltpu.async_copy(tab.at[src_v], rows_v, sem).wait()
        pltpu.sync_copy(rows_v, acc.at[dst_v], add=True)
        return carry

    lax.fori_loop(0, NCHUNK, chunk, 0)
    plsc.subcore_barrier()
    pltpu.sync_copy(acc.at[pl.ds(r0, ZR)], out.at[cid, pl.ds(r0, ZR)])

    @pl.when(sid == 0)
    def _():
        pltpu.sync_copy(acc.at[pl.ds(NS * ZR, ZTAIL)],
                        out.at[cid, pl.ds(NS * ZR, ZTAIL)])


BN = 1000


def _mlp(z, W1, b1, W2, b2):
    t = jnp.maximum(
        jnp.dot(z, W1, preferred_element_type=jnp.float32) + b1, 0.0
    )
    return jnp.dot(t, W2, preferred_element_type=jnp.float32) + b2


def _tc1_body(x_ref, a_ref, W10, b10, W20, b20, W11, b11, W21, b21,
              W12, b12, W22, b22, out1_ref, h2a_ref, h3a_ref):
    z = x_ref[...] + a_ref[0] + a_ref[1]
    out1_ref[...] = _mlp(z, W10[...], b10[...], W20[...], b20[...])
    h2a_ref[...] = _mlp(z, W11[...], b11[...], W21[...], b21[...])
    h3a_ref[...] = _mlp(z, W12[...], b12[...], W22[...], b22[...])


def _tc2_body(h2a_ref, a2_ref, h3a_ref, a3_ref, W11, b11, W21, b21,
              W12, b12, W22, b22, out2_ref, h3b_ref):
    z2 = h2a_ref[...] + a2_ref[0] + a2_ref[1]
    out2_ref[...] = _mlp(z2, W11[...], b11[...], W21[...], b21[...])
    z3 = h3a_ref[...] + a3_ref[0] + a3_ref[1]
    h3b_ref[...] = _mlp(z3, W12[...], b12[...], W22[...], b22[...])


def _tc3_body(h3b_ref, a4_ref, out1_ref, out2_ref, W12, b12, W22, b22,
              Wp_ref, bp_ref, y_ref):
    z = h3b_ref[...] + a4_ref[0] + a4_ref[1]
    out3 = _mlp(z, W12[...], b12[...], W22[...], b22[...])
    Wp = Wp_ref[...]
    y = jnp.dot(out1_ref[...], Wp[0:H], preferred_element_type=jnp.float32)
    y += jnp.dot(out2_ref[...], Wp[H:2 * H], preferred_element_type=jnp.float32)
    y += jnp.dot(out3, Wp[2 * H:3 * H], preferred_element_type=jnp.float32)
    y_ref[...] = y + bp_ref[...]


_row_spec = pl.BlockSpec((BN, H), lambda i: (i, 0))
_agg_spec = pl.BlockSpec((NC, BN, H), lambda i: (0, i, 0))
_w_spec = pl.BlockSpec((H, H), lambda i: (0, 0))
_b_spec = pl.BlockSpec((1, H), lambda i: (0, 0))
_out_nh = jax.ShapeDtypeStruct((N, H), jnp.float32)
_grid = (N // BN,)
_tc_params = pltpu.CompilerParams(dimension_semantics=("arbitrary",))

_tc1 = pl.pallas_call(
    _tc1_body,
    grid=_grid,
    in_specs=[_row_spec, _agg_spec] + [_w_spec, _b_spec] * 6,
    out_specs=[_row_spec] * 3,
    out_shape=[_out_nh] * 3,
    compiler_params=_tc_params,
)

_tc2 = pl.pallas_call(
    _tc2_body,
    grid=_grid,
    in_specs=[_row_spec, _agg_spec, _row_spec, _agg_spec]
    + [_w_spec, _b_spec] * 4,
    out_specs=[_row_spec] * 2,
    out_shape=[_out_nh] * 2,
    compiler_params=_tc_params,
)

_tc3 = pl.pallas_call(
    _tc3_body,
    grid=_grid,
    in_specs=[_row_spec, _agg_spec, _row_spec, _row_spec]
    + [_w_spec, _b_spec] * 2
    + [pl.BlockSpec((3 * H, H), lambda i: (0, 0)), _b_spec],
    out_specs=_row_spec,
    out_shape=_out_nh,
    compiler_params=_tc_params,
)


def kernel(x, edge_index, W1_0, b1_0, W2_0, b2_0, W1_1, b1_1, W2_1, b2_1,
           W1_2, b1_2, W2_2, b2_2, Wp, bp):
    src = edge_index[0]
    dst = edge_index[1]
    zeros = jnp.zeros((N, H), jnp.float32)
    b1_0r, b2_0r = b1_0.reshape(1, H), b2_0.reshape(1, H)
    b1_1r, b2_1r = b1_1.reshape(1, H), b2_1.reshape(1, H)
    b1_2r, b2_2r = b1_2.reshape(1, H), b2_2.reshape(1, H)
    bpr = bp.reshape(1, H)

    agg0 = _segsum(x, src, dst, zeros)
    out1, h2a, h3a = _tc1(x, agg0, W1_0, b1_0r, W2_0, b2_0r,
                          W1_1, b1_1r, W2_1, b2_1r, W1_2, b1_2r, W2_2, b2_2r)
    agg2 = _segsum(h2a, src, dst, zeros)
    agg3a = _segsum(h3a, src, dst, zeros)
    out2, h3b = _tc2(h2a, agg2, h3a, agg3a,
                     W1_1, b1_1r, W2_1, b2_1r, W1_2, b1_2r, W2_2, b2_2r)
    agg3b = _segsum(h3b, src, dst, zeros)
    y = _tc3(h3b, agg3b, out1, out2, W1_2, b1_2r, W2_2, b2_2r, Wp, bpr)
    return y
